# baseline (speedup 1.0000x reference)
import math

import jax
import jax.numpy as jnp
from jax import lax
from jax.experimental import pallas as pl
from jax.experimental.pallas import tpu as pltpu
from jax.experimental.pallas import tpu_sc as plsc

_B = 1048576
_N_LEVELS = 16
_F = 2
_BASE = 16.0
_FINEST = 8192.0
_IMG = 1.0
_b = math.exp((math.log(_FINEST) - math.log(_BASE)) / (_N_LEVELS - 1))

_RES = [float(math.floor(_BASE * _b ** i)) for i in range(_N_LEVELS)]
_GS = [_IMG / r for r in _RES]
_N_USED = [int(r) + 2 for r in _RES]
_SEG_ELEMS = [((2 * n + 7) // 8) * 8 for n in _N_USED]
_OFF = [0]
for _e in _SEG_ELEMS[:-1]:
    _OFF.append(_OFF[-1] + _e)
_T_ELEMS = _OFF[-1] + _SEG_ELEMS[-1]

_NC = 2
_NS = 16
_NW = _NC * _NS
_L = 16

_PER_W = _B // _NW
_BLK = 256
_NB = _PER_W // _BLK
_VPB = _BLK // _L


def _tec_body(x_hbm, tab_hbm, out_hbm, tab_v, x_v0, x_v1, cmp_v, dma_v0,
              dma_v1, sem0, sem1, sx0, sx1):
    wid = lax.axis_index("s") * _NC + lax.axis_index("c")
    base = wid * _PER_W

    pltpu.sync_copy(tab_hbm, tab_v)
    lane = lax.iota(jnp.int32, 16)

    def x_src(ib):
        row0 = base + ib * _BLK
        return x_hbm.at[pl.ds(pl.multiple_of(row0, 8), _BLK)]

    def run_inner(x_v):
        @plsc.parallel_loop(0, _VPB, unroll=8)
        def _loop(j):
            xv = x_v[pl.ds(j * _L, _L)]
            obase = lane * 33 + j * (_L * 33)
            for l in range(_N_LEVELS):
                t = xv * _RES[l]
                li = t.astype(jnp.int32)
                w = t - li.astype(jnp.float32)
                eidx = li * 2 + _OFF[l]
                e00 = plsc.load_gather(tab_v, [eidx])
                e01 = plsc.load_gather(tab_v, [eidx + 1])
                e10 = plsc.load_gather(tab_v, [eidx + 2])
                e11 = plsc.load_gather(tab_v, [eidx + 3])
                c0 = e00 + (e10 - e00) * w
                c1 = e01 + (e11 - e01) * w
                plsc.store_scatter(cmp_v, [obase + (2 * l)], c0)
                plsc.store_scatter(cmp_v, [obase + (2 * l + 1)], c1)

    def repack(dma_v):
        @plsc.parallel_loop(0, _BLK, unroll=32)
        def _rp(p):
            lo = cmp_v[pl.ds(p * 33, _L)]
            hi = cmp_v[pl.ds(p * 33 + 16, _L)]
            dma_v[p, pl.ds(0, _L)] = lo
            dma_v[p, pl.ds(16, _L)] = hi

    def out_slice(row0):
        return out_hbm.at[pl.ds(pl.multiple_of(row0, 8), _BLK), :]

    def dma_src(dma_v):
        return dma_v.at[:, :]

    def half(ib, x_v, sx, dma_v, sem):
        row0 = base + ib * _BLK
        pltpu.make_async_copy(x_src(ib), x_v, sx).wait()
        run_inner(x_v)

        @pl.when(ib + 2 < _NB)
        def _prefetch():
            pltpu.async_copy(x_src(ib + 2), x_v, sx)

        @pl.when(ib >= 2)
        def _wait():
            pltpu.make_async_copy(
                dma_src(dma_v), out_slice(row0 - 2 * _BLK), sem).wait()

        repack(dma_v)
        pltpu.async_copy(dma_src(dma_v), out_slice(row0), sem)

    pltpu.async_copy(x_src(0), x_v0, sx0)
    pltpu.async_copy(x_src(1), x_v1, sx1)

    def pair(i2, carry):
        half(i2 * 2, x_v0, sx0, dma_v0, sem0)
        half(i2 * 2 + 1, x_v1, sx1, dma_v1, sem1)
        return carry

    lax.fori_loop(0, _NB // 2, pair, 0, unroll=False)
    last0 = base + (_NB - 2) * _BLK
    last1 = base + (_NB - 1) * _BLK
    pltpu.make_async_copy(dma_src(dma_v0), out_slice(last0), sem0).wait()
    pltpu.make_async_copy(dma_src(dma_v1), out_slice(last1), sem1).wait()


@jax.jit
def _hash_embed(x_flat, tab_flat):
    mesh = plsc.VectorSubcoreMesh(core_axis_name="c", subcore_axis_name="s")
    fn = pl.kernel(
        _tec_body,
        out_type=jax.ShapeDtypeStruct((_B, 128), jnp.float32),
        mesh=mesh,
        compiler_params=pltpu.CompilerParams(
            needs_layout_passes=False, use_tc_tiling_on_sc=True),
        scratch_types=[
            pltpu.VMEM((_T_ELEMS,), jnp.float32),
            pltpu.VMEM((_BLK,), jnp.float32),
            pltpu.VMEM((_BLK,), jnp.float32),
            pltpu.VMEM((_BLK * 33,), jnp.float32),
            pltpu.VMEM((_BLK, 128), jnp.float32),
            pltpu.VMEM((_BLK, 128), jnp.float32),
            pltpu.SemaphoreType.DMA,
            pltpu.SemaphoreType.DMA,
            pltpu.SemaphoreType.DMA,
            pltpu.SemaphoreType.DMA,
        ],
    )
    return fn(x_flat, tab_flat)


def kernel(x, tables):
    segs = []
    for i in range(_N_LEVELS):
        seg = tables[i][: _N_USED[i]].reshape(-1)
        pad = _SEG_ELEMS[i] - seg.shape[0]
        if pad:
            seg = jnp.concatenate([seg, jnp.zeros((pad,), jnp.float32)])
        segs.append(seg)
    tab_flat = jnp.concatenate(segs)
    out = _hash_embed(x.reshape(-1), tab_flat)
    return out[:, : _N_LEVELS * _F]


# speedup vs baseline: 1.2875x; 1.2875x over previous
import math

import jax
import jax.numpy as jnp
from jax import lax
from jax.experimental import pallas as pl
from jax.experimental.pallas import tpu as pltpu
from jax.experimental.pallas import tpu_sc as plsc

_B = 1048576
_N_LEVELS = 16
_F = 2
_BASE = 16.0
_FINEST = 8192.0
_IMG = 1.0
_b = math.exp((math.log(_FINEST) - math.log(_BASE)) / (_N_LEVELS - 1))

_RES = [float(math.floor(_BASE * _b ** i)) for i in range(_N_LEVELS)]
_GS = [_IMG / r for r in _RES]
_N_USED = [int(r) + 2 for r in _RES]
_SEG_ELEMS = [((2 * n + 7) // 8) * 8 for n in _N_USED]
_OFF = [0]
for _e in _SEG_ELEMS[:-1]:
    _OFF.append(_OFF[-1] + _e)
_T_ELEMS = _OFF[-1] + _SEG_ELEMS[-1]

_NC = 2
_NS = 16
_NW = _NC * _NS
_L = 16

_PER_W = _B // _NW
_BLK = 256
_NB = _PER_W // _BLK
_VPB = _BLK // _L


def _tec_body(x_hbm, tab_hbm, out_hbm, tab_v, x_v0, x_v1, cmp_v, dma_v0,
              dma_v1, sem0, sem1, sx0, sx1):
    wid = lax.axis_index("s") * _NC + lax.axis_index("c")
    base = wid * _PER_W

    pltpu.sync_copy(tab_hbm, tab_v)
    lane = lax.iota(jnp.int32, 16)

    def x_src(ib):
        row0 = base + ib * _BLK
        return x_hbm.at[pl.ds(pl.multiple_of(row0, 8), _BLK)]

    def run_inner(x_v):
        @plsc.parallel_loop(0, _VPB, unroll=4)
        def _loop(j):
            xv = x_v[pl.ds(j * _L, _L)]
            obase = lane * 33 + j * (_L * 33)
            for l in range(_N_LEVELS):
                t = xv * _RES[l]
                li = t.astype(jnp.int32)
                w = t - li.astype(jnp.float32)
                eidx = li * 2 + _OFF[l]
                e00 = plsc.load_gather(tab_v, [eidx])
                e01 = plsc.load_gather(tab_v, [eidx + 1])
                e10 = plsc.load_gather(tab_v, [eidx + 2])
                e11 = plsc.load_gather(tab_v, [eidx + 3])
                c0 = e00 + (e10 - e00) * w
                c1 = e01 + (e11 - e01) * w
                plsc.store_scatter(cmp_v, [obase + (2 * l)], c0)
                plsc.store_scatter(cmp_v, [obase + (2 * l + 1)], c1)

    def repack(dma_v):
        @plsc.parallel_loop(0, _BLK, unroll=16)
        def _rp(p):
            lo = cmp_v[pl.ds(p * 33, _L)]
            hi = cmp_v[pl.ds(p * 33 + 16, _L)]
            dma_v[p, pl.ds(0, _L)] = lo
            dma_v[p, pl.ds(16, _L)] = hi

    def out_slice(row0):
        return out_hbm.at[pl.ds(pl.multiple_of(row0, 8), _BLK), :]

    def dma_src(dma_v):
        return dma_v.at[:, :]

    def half(ib, x_v, sx, dma_v, sem):
        row0 = base + ib * _BLK
        pltpu.make_async_copy(x_src(ib), x_v, sx).wait()
        run_inner(x_v)

        @pl.when(ib + 2 < _NB)
        def _prefetch():
            pltpu.async_copy(x_src(ib + 2), x_v, sx)

        @pl.when(ib >= 2)
        def _wait():
            pltpu.make_async_copy(
                dma_src(dma_v), out_slice(row0 - 2 * _BLK), sem).wait()

        repack(dma_v)
        pltpu.async_copy(dma_src(dma_v), out_slice(row0), sem)

    pltpu.async_copy(x_src(0), x_v0, sx0)
    pltpu.async_copy(x_src(1), x_v1, sx1)

    def pair(i2, carry):
        half(i2 * 2, x_v0, sx0, dma_v0, sem0)
        half(i2 * 2 + 1, x_v1, sx1, dma_v1, sem1)
        return carry

    lax.fori_loop(0, _NB // 2, pair, 0, unroll=False)
    last0 = base + (_NB - 2) * _BLK
    last1 = base + (_NB - 1) * _BLK
    pltpu.make_async_copy(dma_src(dma_v0), out_slice(last0), sem0).wait()
    pltpu.make_async_copy(dma_src(dma_v1), out_slice(last1), sem1).wait()


@jax.jit
def _hash_embed(x_flat, tab_flat):
    mesh = plsc.VectorSubcoreMesh(core_axis_name="c", subcore_axis_name="s")
    fn = pl.kernel(
        _tec_body,
        out_type=jax.ShapeDtypeStruct((_B, 128), jnp.float32),
        mesh=mesh,
        compiler_params=pltpu.CompilerParams(
            needs_layout_passes=False, use_tc_tiling_on_sc=True),
        scratch_types=[
            pltpu.VMEM((_T_ELEMS,), jnp.float32),
            pltpu.VMEM((_BLK,), jnp.float32),
            pltpu.VMEM((_BLK,), jnp.float32),
            pltpu.VMEM((_BLK * 33,), jnp.float32),
            pltpu.VMEM((_BLK, 128), jnp.float32),
            pltpu.VMEM((_BLK, 128), jnp.float32),
            pltpu.SemaphoreType.DMA,
            pltpu.SemaphoreType.DMA,
            pltpu.SemaphoreType.DMA,
            pltpu.SemaphoreType.DMA,
        ],
    )
    return fn(x_flat, tab_flat)


def kernel(x, tables):
    segs = []
    for i in range(_N_LEVELS):
        seg = tables[i][: _N_USED[i]].reshape(-1)
        pad = _SEG_ELEMS[i] - seg.shape[0]
        if pad:
            seg = jnp.concatenate([seg, jnp.zeros((pad,), jnp.float32)])
        segs.append(seg)
    tab_flat = jnp.concatenate(segs)
    out = _hash_embed(x.reshape(-1), tab_flat)
    return out[:, : _N_LEVELS * _F]
